# baseline (speedup 1.0000x reference)
import functools

import jax
import jax.numpy as jnp
from jax import lax
from jax.experimental import pallas as pl
from jax.experimental.pallas import tpu as pltpu
from jax.experimental.pallas import tpu_sc as plsc

_NUM_CORES = 2
_NUM_SUBCORES = 16
_NUM_WORKERS = _NUM_CORES * _NUM_SUBCORES

_TC_COLS = 131072


def _tc_table_body(embt_ref, w1_ref, b1_ref, w2_ref, b2_ref, out_ref):
    blk = embt_ref[...].astype(jnp.bfloat16)
    w1 = w1_ref[...].astype(jnp.bfloat16)
    h = lax.dot_general(w1, blk, (((0,), (0,)), ((), ())),
                        preferred_element_type=jnp.float32)
    h = jnp.maximum(h + b1_ref[...], 0.0).astype(jnp.bfloat16)
    w2 = w2_ref[...].astype(jnp.bfloat16)
    s = lax.dot_general(w2, h, (((0,), (0,)), ((), ())),
                        preferred_element_type=jnp.float32)
    s = jax.nn.sigmoid(s + b2_ref[...])
    out_ref[...] = s[0]


def _make_score_table(emb, W1, b1, W2, b2, v_pad):
    V, D = emb.shape
    H = W1.shape[1]
    embt = emb.T
    grid = pl.cdiv(v_pad, _TC_COLS)
    return pl.pallas_call(
        _tc_table_body,
        grid=(grid,),
        in_specs=[
            pl.BlockSpec((D, _TC_COLS), lambda i: (0, i)),
            pl.BlockSpec((D, H), lambda i: (0, 0)),
            pl.BlockSpec((H, 1), lambda i: (0, 0)),
            pl.BlockSpec((H, 1), lambda i: (0, 0)),
            pl.BlockSpec((1, 1), lambda i: (0, 0)),
        ],
        out_specs=pl.BlockSpec((_TC_COLS,), lambda i: (i,)),
        out_shape=jax.ShapeDtypeStruct((v_pad,), jnp.float32),
    )(embt, W1, b1.reshape(H, 1), W2, b2.reshape(1, 1))


def _sc_gather(table, idx):
    n = idx.shape[0]
    n_per_w = n // _NUM_WORKERS
    mesh = plsc.VectorSubcoreMesh(
        core_axis_name="c", subcore_axis_name="s",
        num_cores=_NUM_CORES, num_subcores=_NUM_SUBCORES)

    @functools.partial(
        pl.kernel,
        mesh=mesh,
        out_type=jax.ShapeDtypeStruct((n,), jnp.float32),
        scratch_types=[
            pltpu.VMEM((n_per_w,), jnp.int32),
            pltpu.VMEM((n_per_w,), jnp.float32),
            pltpu.SemaphoreType.DMA,
        ],
    )
    def gather_kernel(table_hbm, idx_hbm, out_hbm, idx_v, rows_v, sem):
        wid = lax.axis_index("s") * _NUM_CORES + lax.axis_index("c")
        base = wid * n_per_w
        pltpu.sync_copy(idx_hbm.at[pl.ds(base, n_per_w)], idx_v)
        pltpu.async_copy(table_hbm.at[idx_v], rows_v, sem).wait()
        pltpu.sync_copy(rows_v, out_hbm.at[pl.ds(base, n_per_w)])

    return gather_kernel(table, idx)


def kernel(x, emb, W1, b1, W2, b2):
    B, L = x.shape
    n = B * L
    assert n % _NUM_WORKERS == 0

    v_pad = 1000064
    table = _make_score_table(emb, W1, b1, W2, b2, v_pad)
    idx = x.T.reshape(n)
    scores = _sc_gather(table, idx)
    return scores.reshape(L, B, 1).transpose(1, 0, 2)
